# baseline (speedup 1.0000x reference)
import functools

import jax
import jax.numpy as jnp
import numpy as np
from jax import lax
from jax.experimental import pallas as pl
from jax.experimental.pallas import tpu as pltpu
from jax.experimental.pallas import tpu_sc as plsc

NW = 32
LANES = 16


def _mesh():
    return plsc.VectorSubcoreMesh(core_axis_name="c", subcore_axis_name="s")


def _wid():
    return lax.axis_index("s") * 2 + lax.axis_index("c")


def _gather_rows(table, idx, chunk, nbuf, tc_tiling=True):
    n = idx.shape[0]
    d = table.shape[1]
    per = n // NW
    assert per * NW == n and per % chunk == 0
    nchunks = per // chunk
    assert nchunks % nbuf == 0 and chunk % 8 == 0 and chunk <= 128
    ngroups = nchunks // nbuf

    @functools.partial(
        pl.kernel,
        mesh=_mesh(),
        compiler_params=pltpu.CompilerParams(use_tc_tiling_on_sc=tc_tiling),
        out_type=jax.ShapeDtypeStruct((n, d), table.dtype),
        scratch_types=[
            pltpu.VMEM((per,), jnp.int32),
            pltpu.VMEM((2 * nbuf, chunk, d), table.dtype),
            pltpu.SemaphoreType.DMA,
            pltpu.SemaphoreType.DMA,
        ],
    )
    def k(table_hbm, idx_hbm, out_hbm, idx_v, bufs, gsem, osem):
        base = _wid() * per
        pltpu.sync_copy(idx_hbm.at[pl.ds(base, per)], idx_v)

        def group(g, carry):
            bank = (g % 2) * nbuf

            @pl.when(g > 1)
            def _():
                for b in range(nbuf):
                    pltpu.make_async_copy(
                        out_hbm.at[pl.ds(base, chunk), :], bufs.at[0], osem
                    ).wait()

            hs = []
            for b in range(nbuf):
                c = g * nbuf + b
                hs.append(
                    pltpu.async_copy(
                        table_hbm.at[idx_v.at[pl.ds(c * chunk, chunk)]],
                        bufs.at[bank + b],
                        gsem,
                    )
                )
            for h in hs:
                h.wait()
            for b in range(nbuf):
                c = g * nbuf + b
                pltpu.async_copy(
                    bufs.at[bank + b],
                    out_hbm.at[pl.ds(base + c * chunk, chunk), :],
                    osem,
                )
            return carry

        lax.fori_loop(0, ngroups, group, 0)
        for g in range(min(2, ngroups)):
            for b in range(nbuf):
                pltpu.make_async_copy(
                    out_hbm.at[pl.ds(base, chunk), :], bufs.at[0], osem
                ).wait()

    return k(table, idx)


def _scatter_update(nf_pad, nodes, h1):
    vp, d = nf_pad.shape
    n3 = nodes.shape[0]
    rng = vp // NW
    assert rng * NW == vp and rng % LANES == 0 and n3 % LANES == 0
    nwin = rng // LANES

    @functools.partial(
        pl.kernel,
        mesh=_mesh(),
        compiler_params=pltpu.CompilerParams(
            needs_layout_passes=False, use_tc_tiling_on_sc=False),
        out_type=jax.ShapeDtypeStruct((vp, d), jnp.float32),
        scratch_types=[
            pltpu.VMEM((n3,), jnp.int32),
            pltpu.VMEM((rng,), jnp.int32),
            pltpu.VMEM((rng, d), jnp.float32),
            pltpu.SemaphoreType.DMA,
            pltpu.SemaphoreType.DMA,
            pltpu.SemaphoreType.DMA,
        ],
    )
    def k(nf_hbm, nodes_hbm, h1_hbm, out_hbm, nodes_v, lp, rows, csem, gsem, ssem):
        lo = _wid() * rng
        ch = pltpu.async_copy(
            nf_hbm.at[pl.ds(lo, rng), :], out_hbm.at[pl.ds(lo, rng), :], csem
        )
        pltpu.sync_copy(nodes_hbm, nodes_v)
        neg1 = jnp.full((LANES,), -1, jnp.int32)
        for w in range(nwin):
            lp[pl.ds(w * LANES, LANES)] = neg1
        iota = lax.iota(jnp.int32, LANES)

        def scan(i, carry):
            vv = nodes_v[pl.ds(i * LANES, LANES)]
            bvec = i * LANES + iota
            m = (vv >= lo) & (vv < lo + rng)
            idx = jnp.where(m, vv - lo, 0)
            plsc.store_scatter(lp, [idx], bvec, mask=m)
            return carry

        lax.fori_loop(0, n3 // LANES, scan, 0)

        dsts = []
        ghs = []
        for w in range(nwin):
            mv = lp[pl.ds(w * LANES, LANES)]
            msk = mv >= 0
            src = jnp.where(msk, mv, 0)
            dsts.append(jnp.where(msk, lo + w * LANES + iota, (vp - LANES) + iota))
            ghs.append(
                pltpu.async_copy(
                    h1_hbm.at[src], rows.at[pl.ds(w * LANES, LANES)], gsem
                )
            )
        for h in ghs:
            h.wait()
        ch.wait()
        shs = []
        for w in range(nwin):
            shs.append(
                pltpu.async_copy(
                    rows.at[pl.ds(w * LANES, LANES)], out_hbm.at[dsts[w]], ssem
                )
            )
        for h in shs:
            h.wait()

    return k(nf_pad, nodes, h1)


def _tc_layer(h, ts2, hn_t, ef_t, nbr, qwT, kwnT, kweT, vwnT, vweT, bq, bk, bv,
              owT, ob, tw, tb):
    n3, d = h.shape
    nbrk = hn_t.shape[0]
    de = ef_t.shape[2]
    qb = 256
    assert n3 % qb == 0
    dh = d // 2
    scale = 1.0 / float(np.sqrt(dh))

    def body(h_ref, ts_ref, hn_ref, ef_ref, nbr_ref, qw_ref, kwn_ref, kwe_ref,
             vwn_ref, vwe_ref, bq_ref, bk_ref, bv_ref, ow_ref, ob_ref, tw_ref,
             tb_ref, out_ref):
        tenc = jnp.cos(ts_ref[...] * tw_ref[...] + tb_ref[...])
        qw16 = qw_ref[...].astype(jnp.bfloat16)
        q = (jnp.dot(h_ref[...], qw16, preferred_element_type=jnp.float32)
             + jnp.dot(tenc, qw_ref[...], preferred_element_type=jnp.float32)
             + bq_ref[...])

        nbrv = nbr_ref[...]
        pmf = jnp.where(nbrv == 0, 1.0, 0.0)
        invalf = jnp.where(
            jnp.sum(pmf, axis=1, keepdims=True) >= float(nbrk), 1.0, 0.0)
        colzero = jnp.where(
            lax.broadcasted_iota(jnp.int32, pmf.shape, 1) == 0, 1.0, 0.0)
        pmf = pmf * (1.0 - colzero * invalf)

        kwn = kwn_ref[...]
        kwn16 = kwn.astype(jnp.bfloat16)
        kwe = kwe_ref[...]
        tk = (jnp.dot(tenc, kwn, preferred_element_type=jnp.float32)
              + bk_ref[...])
        s0s, s1s = [], []
        for n in range(nbrk):
            kn = (jnp.dot(hn_ref[n], kwn16, preferred_element_type=jnp.float32)
                  + jnp.dot(ef_ref[n], kwe, preferred_element_type=jnp.float32)
                  + tk)
            qk = q * kn
            s0s.append(jnp.sum(qk[:, :dh], axis=1, keepdims=True))
            s1s.append(jnp.sum(qk[:, dh:], axis=1, keepdims=True))
        s0 = jnp.concatenate(s0s, axis=1) * scale
        s1 = jnp.concatenate(s1s, axis=1) * scale
        s0 = s0 * (1.0 - pmf) + (-1e9) * pmf
        s1 = s1 * (1.0 - pmf) + (-1e9) * pmf

        def smax(s):
            m = jnp.max(s, axis=1, keepdims=True)
            e = jnp.exp(s - m)
            return e / jnp.sum(e, axis=1, keepdims=True)

        a0 = smax(s0)
        a1 = smax(s1)

        vwn = vwn_ref[...]
        vwn16 = vwn.astype(jnp.bfloat16)
        vwe = vwe_ref[...]
        tv = (jnp.dot(tenc, vwn, preferred_element_type=jnp.float32)
              + bv_ref[...])
        lane = lax.broadcasted_iota(jnp.int32, (1, d), 1)
        m0 = (lane < dh).astype(jnp.float32)
        m1 = 1.0 - m0
        acc = jnp.zeros((qb, d), jnp.float32)
        for n in range(nbrk):
            vn = (jnp.dot(hn_ref[n], vwn16, preferred_element_type=jnp.float32)
                  + jnp.dot(ef_ref[n], vwe, preferred_element_type=jnp.float32)
                  + tv)
            w = a0[:, n:n + 1] * m0 + a1[:, n:n + 1] * m1
            acc = acc + w * vn
        out_ref[...] = (jnp.dot(acc, ow_ref[...], preferred_element_type=jnp.float32)
                        + ob_ref[...])

    full = lambda i: (0, 0)
    return pl.pallas_call(
        body,
        grid=(n3 // qb,),
        in_specs=[
            pl.BlockSpec((qb, d), lambda i: (i, 0)),
            pl.BlockSpec((qb, 1), lambda i: (i, 0)),
            pl.BlockSpec((nbrk, qb, d), lambda i: (0, i, 0)),
            pl.BlockSpec((nbrk, qb, de), lambda i: (0, i, 0)),
            pl.BlockSpec((qb, nbrk), lambda i: (i, 0)),
            pl.BlockSpec((d, d), full),
            pl.BlockSpec((d, d), full),
            pl.BlockSpec((de, d), full),
            pl.BlockSpec((d, d), full),
            pl.BlockSpec((de, d), full),
            pl.BlockSpec((1, d), full),
            pl.BlockSpec((1, d), full),
            pl.BlockSpec((1, d), full),
            pl.BlockSpec((d, d), full),
            pl.BlockSpec((1, d), full),
            pl.BlockSpec((1, d), full),
            pl.BlockSpec((1, d), full),
        ],
        out_specs=pl.BlockSpec((qb, d), lambda i: (i, 0)),
        out_shape=jax.ShapeDtypeStruct((n3, d), jnp.float32),
    )(h, ts2, hn_t, ef_t, nbr, qwT, kwnT, kweT, vwnT, vweT, bq, bk, bv, owT,
      ob, tw, tb)


def kernel(source_nodes, destination_nodes, negative_nodes, edge_times,
           edge_idxs, neighbors, nbr_edge_idxs, n_neighbors, node_features,
           edge_features, time_w, time_b, q_w, k_w, v_w, in_b, out_w, out_b):
    b = source_nodes.shape[0]
    n3 = 3 * b
    nbrk = neighbors.shape[1]
    v, d = node_features.shape
    de = edge_features.shape[1]

    nodes = jnp.concatenate([source_nodes, destination_nodes, negative_nodes])
    nodes = nodes.astype(jnp.int32)
    ts2 = jnp.concatenate([edge_times, edge_times, edge_times])[:, None]

    vp = ((v + NW * LANES - 1) // (NW * LANES)) * NW * LANES
    nf_pad = jnp.concatenate(
        [node_features, jnp.zeros((vp - v, d), jnp.float32)], axis=0)

    nbr_t_flat = neighbors.astype(jnp.int32).T.reshape(-1)
    idx_all = jnp.concatenate([nodes, nbr_t_flat])
    ef_idx = nbr_edge_idxs.astype(jnp.int32).T.reshape(-1)

    qwT = q_w.T
    kwnT = k_w[:, :d].T
    kweT = k_w[:, d:].T
    vwnT = v_w[:, :d].T
    vweT = v_w[:, d:].T
    bq = in_b[:d][None, :]
    bk = in_b[d:2 * d][None, :]
    bv = in_b[2 * d:][None, :]
    ob = out_b[None, :]
    tw = time_w[:, 0][None, :]
    tb = time_b[None, :]

    nfv = lax.bitcast_convert_type(
        nf_pad.astype(jnp.bfloat16).reshape(vp, d // 2, 2), jnp.float32)

    g1v = _gather_rows(nfv, idx_all, chunk=112, nbuf=4, tc_tiling=False)
    efg = _gather_rows(edge_features, ef_idx, chunk=96, nbuf=4, tc_tiling=False)
    g1b = lax.bitcast_convert_type(g1v, jnp.bfloat16).reshape(n3 * (1 + nbrk), d)
    h0 = g1b[:n3]
    hn_t = g1b[n3:].reshape(nbrk, n3, d)
    ef_t = efg.reshape(nbrk, n3, de)

    h1 = _tc_layer(h0, ts2, hn_t, ef_t, neighbors, qwT, kwnT, kweT, vwnT,
                   vweT, bq, bk, bv, out_w.T, ob, tw, tb)

    h1v = lax.bitcast_convert_type(
        h1.astype(jnp.bfloat16).reshape(n3, d // 2, 2), jnp.float32)
    nf2v = _scatter_update(nfv, nodes, h1v)
    g2v = _gather_rows(nf2v, idx_all, chunk=112, nbuf=4, tc_tiling=False)
    g2b = lax.bitcast_convert_type(g2v, jnp.bfloat16).reshape(n3 * (1 + nbrk), d)
    h0b = g2b[:n3]
    hn2_t = g2b[n3:].reshape(nbrk, n3, d)

    h2 = _tc_layer(h0b, ts2, hn2_t, ef_t, neighbors, qwT, kwnT, kweT, vwnT,
                   vweT, bq, bk, bv, out_w.T, ob, tw, tb)

    return (h2[:b], h2[b:2 * b], h2[2 * b:])


# speedup vs baseline: 1.5063x; 1.5063x over previous
import functools

import jax
import jax.numpy as jnp
import numpy as np
from jax import lax
from jax.experimental import pallas as pl
from jax.experimental.pallas import tpu as pltpu
from jax.experimental.pallas import tpu_sc as plsc

NW = 32
LANES = 16


def _mesh():
    return plsc.VectorSubcoreMesh(core_axis_name="c", subcore_axis_name="s")


def _wid():
    return lax.axis_index("s") * 2 + lax.axis_index("c")


def _gather_rows(table, idx, chunk, nbuf, tc_tiling=True):
    n = idx.shape[0]
    d = table.shape[1]
    per = n // NW
    assert per * NW == n and per % chunk == 0
    nchunks = per // chunk
    assert nchunks % nbuf == 0 and chunk % 8 == 0 and chunk <= 128
    ngroups = nchunks // nbuf

    @functools.partial(
        pl.kernel,
        mesh=_mesh(),
        compiler_params=pltpu.CompilerParams(use_tc_tiling_on_sc=tc_tiling),
        out_type=jax.ShapeDtypeStruct((n, d), table.dtype),
        scratch_types=[
            pltpu.VMEM((per,), jnp.int32),
            pltpu.VMEM((2 * nbuf, chunk, d), table.dtype),
            pltpu.SemaphoreType.DMA,
            pltpu.SemaphoreType.DMA,
        ],
    )
    def k(table_hbm, idx_hbm, out_hbm, idx_v, bufs, gsem, osem):
        base = _wid() * per
        pltpu.sync_copy(idx_hbm.at[pl.ds(base, per)], idx_v)

        def group(g, carry):
            bank = (g % 2) * nbuf

            @pl.when(g > 1)
            def _():
                for b in range(nbuf):
                    pltpu.make_async_copy(
                        out_hbm.at[pl.ds(base, chunk), :], bufs.at[0], osem
                    ).wait()

            hs = []
            for b in range(nbuf):
                c = g * nbuf + b
                hs.append(
                    pltpu.async_copy(
                        table_hbm.at[idx_v.at[pl.ds(c * chunk, chunk)]],
                        bufs.at[bank + b],
                        gsem,
                    )
                )
            for h in hs:
                h.wait()
            for b in range(nbuf):
                c = g * nbuf + b
                pltpu.async_copy(
                    bufs.at[bank + b],
                    out_hbm.at[pl.ds(base + c * chunk, chunk), :],
                    osem,
                )
            return carry

        lax.fori_loop(0, ngroups, group, 0)
        for g in range(min(2, ngroups)):
            for b in range(nbuf):
                pltpu.make_async_copy(
                    out_hbm.at[pl.ds(base, chunk), :], bufs.at[0], osem
                ).wait()

    return k(table, idx)


def _scatter_update(nf_pad, nodes, h1):
    vp, d = nf_pad.shape
    n3 = nodes.shape[0]
    rng = vp // NW
    assert rng * NW == vp and rng % LANES == 0 and n3 % LANES == 0
    nwin = rng // LANES

    @functools.partial(
        pl.kernel,
        mesh=_mesh(),
        compiler_params=pltpu.CompilerParams(
            needs_layout_passes=False, use_tc_tiling_on_sc=False),
        out_type=jax.ShapeDtypeStruct((vp, d), jnp.float32),
        scratch_types=[
            pltpu.VMEM((n3,), jnp.int32),
            pltpu.VMEM((rng,), jnp.int32),
            pltpu.VMEM((rng, d), jnp.float32),
            pltpu.SemaphoreType.DMA,
            pltpu.SemaphoreType.DMA,
            pltpu.SemaphoreType.DMA,
        ],
    )
    def k(nf_hbm, nodes_hbm, h1_hbm, out_hbm, nodes_v, lp, rows, csem, gsem, ssem):
        lo = _wid() * rng
        ch = pltpu.async_copy(
            nf_hbm.at[pl.ds(lo, rng), :], out_hbm.at[pl.ds(lo, rng), :], csem
        )
        pltpu.sync_copy(nodes_hbm, nodes_v)
        neg1 = jnp.full((LANES,), -1, jnp.int32)
        for w in range(nwin):
            lp[pl.ds(w * LANES, LANES)] = neg1
        iota = lax.iota(jnp.int32, LANES)

        def scan(i, carry):
            vv = nodes_v[pl.ds(i * LANES, LANES)]
            bvec = i * LANES + iota
            m = (vv >= lo) & (vv < lo + rng)
            idx = jnp.where(m, vv - lo, 0)
            plsc.store_scatter(lp, [idx], bvec, mask=m)
            return carry

        lax.fori_loop(0, n3 // LANES, scan, 0)

        dsts = []
        ghs = []
        for w in range(nwin):
            mv = lp[pl.ds(w * LANES, LANES)]
            msk = mv >= 0
            src = jnp.where(msk, mv, 0)
            dsts.append(jnp.where(msk, lo + w * LANES + iota, (vp - LANES) + iota))
            ghs.append(
                pltpu.async_copy(
                    h1_hbm.at[src], rows.at[pl.ds(w * LANES, LANES)], gsem
                )
            )
        for h in ghs:
            h.wait()
        ch.wait()
        shs = []
        for w in range(nwin):
            shs.append(
                pltpu.async_copy(
                    rows.at[pl.ds(w * LANES, LANES)], out_hbm.at[dsts[w]], ssem
                )
            )
        for h in shs:
            h.wait()

    return k(nf_pad, nodes, h1)


def _tc_layer(h, ts2, hn_t, ef_t, nbr, qwT, kwnT, kweT, vwnT, vweT, bq, bk, bv,
              owT, ob, tw, tb):
    n3, d = h.shape
    nbrk = hn_t.shape[0]
    de = ef_t.shape[2]
    qb = 256
    assert n3 % qb == 0
    dh = d // 2
    scale = 1.0 / float(np.sqrt(dh))

    def body(h_ref, ts_ref, hn_ref, ef_ref, nbr_ref, qw_ref, kwn_ref, kwe_ref,
             vwn_ref, vwe_ref, bq_ref, bk_ref, bv_ref, ow_ref, ob_ref, tw_ref,
             tb_ref, out_ref):
        tenc = jnp.cos(ts_ref[...] * tw_ref[...] + tb_ref[...])
        qw16 = qw_ref[...].astype(jnp.bfloat16)
        q = (jnp.dot(h_ref[...].astype(jnp.bfloat16), qw16,
                     preferred_element_type=jnp.float32)
             + jnp.dot(tenc, qw_ref[...], preferred_element_type=jnp.float32)
             + bq_ref[...])

        nbrv = nbr_ref[...]
        pmf = jnp.where(nbrv == 0, 1.0, 0.0)
        invalf = jnp.where(
            jnp.sum(pmf, axis=1, keepdims=True) >= float(nbrk), 1.0, 0.0)
        colzero = jnp.where(
            lax.broadcasted_iota(jnp.int32, pmf.shape, 1) == 0, 1.0, 0.0)
        pmf = pmf * (1.0 - colzero * invalf)

        kwn = kwn_ref[...]
        kwn16 = kwn.astype(jnp.bfloat16)
        kwe = kwe_ref[...]
        tk = (jnp.dot(tenc, kwn, preferred_element_type=jnp.float32)
              + bk_ref[...])
        s0s, s1s = [], []
        for n in range(nbrk):
            kn = (jnp.dot(hn_ref[n].astype(jnp.bfloat16), kwn16,
                          preferred_element_type=jnp.float32)
                  + jnp.dot(ef_ref[n], kwe, preferred_element_type=jnp.float32)
                  + tk)
            qk = q * kn
            s0s.append(jnp.sum(qk[:, :dh], axis=1, keepdims=True))
            s1s.append(jnp.sum(qk[:, dh:], axis=1, keepdims=True))
        s0 = jnp.concatenate(s0s, axis=1) * scale
        s1 = jnp.concatenate(s1s, axis=1) * scale
        s0 = s0 * (1.0 - pmf) + (-1e9) * pmf
        s1 = s1 * (1.0 - pmf) + (-1e9) * pmf

        def smax(s):
            m = jnp.max(s, axis=1, keepdims=True)
            e = jnp.exp(s - m)
            return e / jnp.sum(e, axis=1, keepdims=True)

        a0 = smax(s0)
        a1 = smax(s1)

        vwn = vwn_ref[...]
        vwn16 = vwn.astype(jnp.bfloat16)
        vwe = vwe_ref[...]
        tv = (jnp.dot(tenc, vwn, preferred_element_type=jnp.float32)
              + bv_ref[...])
        lane = lax.broadcasted_iota(jnp.int32, (1, d), 1)
        m0 = (lane < dh).astype(jnp.float32)
        m1 = 1.0 - m0
        acc = jnp.zeros((qb, d), jnp.float32)
        for n in range(nbrk):
            vn = (jnp.dot(hn_ref[n].astype(jnp.bfloat16), vwn16,
                          preferred_element_type=jnp.float32)
                  + jnp.dot(ef_ref[n], vwe, preferred_element_type=jnp.float32)
                  + tv)
            w = a0[:, n:n + 1] * m0 + a1[:, n:n + 1] * m1
            acc = acc + w * vn
        out_ref[...] = (jnp.dot(acc, ow_ref[...], preferred_element_type=jnp.float32)
                        + ob_ref[...])

    full = lambda i: (0, 0)
    return pl.pallas_call(
        body,
        grid=(n3 // qb,),
        in_specs=[
            pl.BlockSpec((qb, d), lambda i: (i, 0)),
            pl.BlockSpec((qb, 1), lambda i: (i, 0)),
            pl.BlockSpec((nbrk, qb, d), lambda i: (0, i, 0)),
            pl.BlockSpec((nbrk, qb, de), lambda i: (0, i, 0)),
            pl.BlockSpec((qb, nbrk), lambda i: (i, 0)),
            pl.BlockSpec((d, d), full),
            pl.BlockSpec((d, d), full),
            pl.BlockSpec((de, d), full),
            pl.BlockSpec((d, d), full),
            pl.BlockSpec((de, d), full),
            pl.BlockSpec((1, d), full),
            pl.BlockSpec((1, d), full),
            pl.BlockSpec((1, d), full),
            pl.BlockSpec((d, d), full),
            pl.BlockSpec((1, d), full),
            pl.BlockSpec((1, d), full),
            pl.BlockSpec((1, d), full),
        ],
        out_specs=pl.BlockSpec((qb, d), lambda i: (i, 0)),
        out_shape=jax.ShapeDtypeStruct((n3, d), jnp.float32),
    )(h, ts2, hn_t, ef_t, nbr, qwT, kwnT, kweT, vwnT, vweT, bq, bk, bv, owT,
      ob, tw, tb)


def kernel(source_nodes, destination_nodes, negative_nodes, edge_times,
           edge_idxs, neighbors, nbr_edge_idxs, n_neighbors, node_features,
           edge_features, time_w, time_b, q_w, k_w, v_w, in_b, out_w, out_b):
    b = source_nodes.shape[0]
    n3 = 3 * b
    nbrk = neighbors.shape[1]
    v, d = node_features.shape
    de = edge_features.shape[1]

    nodes = jnp.concatenate([source_nodes, destination_nodes, negative_nodes])
    nodes = nodes.astype(jnp.int32)
    ts2 = jnp.concatenate([edge_times, edge_times, edge_times])[:, None]

    vp = ((v + NW * LANES - 1) // (NW * LANES)) * NW * LANES
    nf_pad = jnp.concatenate(
        [node_features, jnp.zeros((vp - v, d), jnp.float32)], axis=0)

    nbr_t_flat = neighbors.astype(jnp.int32).T.reshape(-1)
    idx_all = jnp.concatenate([nodes, nbr_t_flat])
    ef_idx = nbr_edge_idxs.astype(jnp.int32).T.reshape(-1)

    qwT = q_w.T
    kwnT = k_w[:, :d].T
    kweT = k_w[:, d:].T
    vwnT = v_w[:, :d].T
    vweT = v_w[:, d:].T
    bq = in_b[:d][None, :]
    bk = in_b[d:2 * d][None, :]
    bv = in_b[2 * d:][None, :]
    ob = out_b[None, :]
    tw = time_w[:, 0][None, :]
    tb = time_b[None, :]

    g1 = _gather_rows(nf_pad, idx_all, chunk=112, nbuf=4)
    efg = _gather_rows(edge_features, ef_idx, chunk=96, nbuf=4, tc_tiling=False)
    h0 = g1[:n3]
    hn_t = g1[n3:].reshape(nbrk, n3, d)
    ef_t = efg.reshape(nbrk, n3, de)

    h1 = _tc_layer(h0, ts2, hn_t, ef_t, neighbors, qwT, kwnT, kweT, vwnT,
                   vweT, bq, bk, bv, out_w.T, ob, tw, tb)

    nf2 = _scatter_update(nf_pad, nodes, h1)
    g2 = _gather_rows(nf2, idx_all, chunk=112, nbuf=4)
    h0b = g2[:n3]
    hn2_t = g2[n3:].reshape(nbrk, n3, d)

    h2 = _tc_layer(h0b, ts2, hn2_t, ef_t, neighbors, qwT, kwnT, kweT, vwnT,
                   vweT, bq, bk, bv, out_w.T, ob, tw, tb)

    return (h2[:b], h2[b:2 * b], h2[2 * b:])


# speedup vs baseline: 1.5074x; 1.0007x over previous
import functools

import jax
import jax.numpy as jnp
import numpy as np
from jax import lax
from jax.experimental import pallas as pl
from jax.experimental.pallas import tpu as pltpu
from jax.experimental.pallas import tpu_sc as plsc

NW = 32
LANES = 16


def _mesh():
    return plsc.VectorSubcoreMesh(core_axis_name="c", subcore_axis_name="s")


def _wid():
    return lax.axis_index("s") * 2 + lax.axis_index("c")


def _gather_rows(table, idx, chunk, nbuf, tc_tiling=True):
    n = idx.shape[0]
    d = table.shape[1]
    per = n // NW
    assert per * NW == n and per % chunk == 0
    nchunks = per // chunk
    assert nchunks % nbuf == 0 and chunk % 8 == 0 and chunk <= 128
    ngroups = nchunks // nbuf

    @functools.partial(
        pl.kernel,
        mesh=_mesh(),
        compiler_params=pltpu.CompilerParams(use_tc_tiling_on_sc=tc_tiling),
        out_type=jax.ShapeDtypeStruct((n, d), table.dtype),
        scratch_types=[
            pltpu.VMEM((per,), jnp.int32),
            pltpu.VMEM((2 * nbuf, chunk, d), table.dtype),
            pltpu.SemaphoreType.DMA,
            pltpu.SemaphoreType.DMA,
        ],
    )
    def k(table_hbm, idx_hbm, out_hbm, idx_v, bufs, gsem, osem):
        base = _wid() * per
        pltpu.sync_copy(idx_hbm.at[pl.ds(base, per)], idx_v)

        def group(g, carry):
            bank = (g % 2) * nbuf

            @pl.when(g > 1)
            def _():
                for b in range(nbuf):
                    pltpu.make_async_copy(
                        out_hbm.at[pl.ds(base, chunk), :], bufs.at[0], osem
                    ).wait()

            hs = []
            for b in range(nbuf):
                c = g * nbuf + b
                hs.append(
                    pltpu.async_copy(
                        table_hbm.at[idx_v.at[pl.ds(c * chunk, chunk)]],
                        bufs.at[bank + b],
                        gsem,
                    )
                )
            for h in hs:
                h.wait()
            for b in range(nbuf):
                c = g * nbuf + b
                pltpu.async_copy(
                    bufs.at[bank + b],
                    out_hbm.at[pl.ds(base + c * chunk, chunk), :],
                    osem,
                )
            return carry

        lax.fori_loop(0, ngroups, group, 0)
        for g in range(min(2, ngroups)):
            for b in range(nbuf):
                pltpu.make_async_copy(
                    out_hbm.at[pl.ds(base, chunk), :], bufs.at[0], osem
                ).wait()

    return k(table, idx)


def _scatter_update(nf_pad, nodes, h1):
    vp, d = nf_pad.shape
    n3 = nodes.shape[0]
    rng = vp // NW
    assert rng * NW == vp and rng % LANES == 0 and n3 % LANES == 0
    nwin = rng // LANES

    @functools.partial(
        pl.kernel,
        mesh=_mesh(),
        compiler_params=pltpu.CompilerParams(needs_layout_passes=False),
        out_type=jax.ShapeDtypeStruct((vp, d), nf_pad.dtype),
        scratch_types=[
            pltpu.VMEM((n3,), jnp.int32),
            pltpu.VMEM((rng,), jnp.int32),
            pltpu.VMEM((rng, d), nf_pad.dtype),
            pltpu.SemaphoreType.DMA,
            pltpu.SemaphoreType.DMA,
            pltpu.SemaphoreType.DMA,
        ],
    )
    def k(nf_hbm, nodes_hbm, h1_hbm, out_hbm, nodes_v, lp, rows, csem, gsem, ssem):
        lo = _wid() * rng
        ch = pltpu.async_copy(
            nf_hbm.at[pl.ds(lo, rng), :], out_hbm.at[pl.ds(lo, rng), :], csem
        )
        pltpu.sync_copy(nodes_hbm, nodes_v)
        neg1 = jnp.full((LANES,), -1, jnp.int32)
        for w in range(nwin):
            lp[pl.ds(w * LANES, LANES)] = neg1
        iota = lax.iota(jnp.int32, LANES)

        def scan(i, carry):
            vv = nodes_v[pl.ds(i * LANES, LANES)]
            bvec = i * LANES + iota
            m = (vv >= lo) & (vv < lo + rng)
            idx = jnp.where(m, vv - lo, 0)
            plsc.store_scatter(lp, [idx], bvec, mask=m)
            return carry

        lax.fori_loop(0, n3 // LANES, scan, 0)

        dsts = []
        ghs = []
        for w in range(nwin):
            mv = lp[pl.ds(w * LANES, LANES)]
            msk = mv >= 0
            src = jnp.where(msk, mv, 0)
            dsts.append(jnp.where(msk, lo + w * LANES + iota, (vp - LANES) + iota))
            ghs.append(
                pltpu.async_copy(
                    h1_hbm.at[src], rows.at[pl.ds(w * LANES, LANES)], gsem
                )
            )
        for h in ghs:
            h.wait()
        ch.wait()
        shs = []
        for w in range(nwin):
            shs.append(
                pltpu.async_copy(
                    rows.at[pl.ds(w * LANES, LANES)], out_hbm.at[dsts[w]], ssem
                )
            )
        for h in shs:
            h.wait()

    return k(nf_pad, nodes, h1)


def _tc_layer(h, ts2, hn_t, ef_t, nbr, qwT, kwnT, kweT, vwnT, vweT, bq, bk, bv,
              owT, ob, tw, tb, out_dtype=jnp.float32):
    n3, d = h.shape
    nbrk = hn_t.shape[0]
    de = ef_t.shape[2]
    qb = 256
    assert n3 % qb == 0
    dh = d // 2
    scale = 1.0 / float(np.sqrt(dh))

    def body(h_ref, ts_ref, hn_ref, ef_ref, nbr_ref, qw_ref, kwn_ref, kwe_ref,
             vwn_ref, vwe_ref, bq_ref, bk_ref, bv_ref, ow_ref, ob_ref, tw_ref,
             tb_ref, out_ref):
        tenc = jnp.cos(ts_ref[...] * tw_ref[...] + tb_ref[...])
        qw16 = qw_ref[...].astype(jnp.bfloat16)
        q = (jnp.dot(h_ref[...].astype(jnp.bfloat16), qw16,
                     preferred_element_type=jnp.float32)
             + jnp.dot(tenc, qw_ref[...], preferred_element_type=jnp.float32)
             + bq_ref[...])

        nbrv = nbr_ref[...]
        pmf = jnp.where(nbrv == 0, 1.0, 0.0)
        invalf = jnp.where(
            jnp.sum(pmf, axis=1, keepdims=True) >= float(nbrk), 1.0, 0.0)
        colzero = jnp.where(
            lax.broadcasted_iota(jnp.int32, pmf.shape, 1) == 0, 1.0, 0.0)
        pmf = pmf * (1.0 - colzero * invalf)

        kwn = kwn_ref[...]
        kwn16 = kwn.astype(jnp.bfloat16)
        kwe = kwe_ref[...]
        tk = (jnp.dot(tenc, kwn, preferred_element_type=jnp.float32)
              + bk_ref[...])
        s0s, s1s = [], []
        for n in range(nbrk):
            kn = (jnp.dot(hn_ref[n].astype(jnp.bfloat16), kwn16,
                          preferred_element_type=jnp.float32)
                  + jnp.dot(ef_ref[n], kwe, preferred_element_type=jnp.float32)
                  + tk)
            qk = q * kn
            s0s.append(jnp.sum(qk[:, :dh], axis=1, keepdims=True))
            s1s.append(jnp.sum(qk[:, dh:], axis=1, keepdims=True))
        s0 = jnp.concatenate(s0s, axis=1) * scale
        s1 = jnp.concatenate(s1s, axis=1) * scale
        s0 = s0 * (1.0 - pmf) + (-1e9) * pmf
        s1 = s1 * (1.0 - pmf) + (-1e9) * pmf

        def smax(s):
            m = jnp.max(s, axis=1, keepdims=True)
            e = jnp.exp(s - m)
            return e / jnp.sum(e, axis=1, keepdims=True)

        a0 = smax(s0)
        a1 = smax(s1)

        vwn = vwn_ref[...]
        vwn16 = vwn.astype(jnp.bfloat16)
        vwe = vwe_ref[...]
        tv = (jnp.dot(tenc, vwn, preferred_element_type=jnp.float32)
              + bv_ref[...])
        lane = lax.broadcasted_iota(jnp.int32, (1, d), 1)
        m0 = (lane < dh).astype(jnp.float32)
        m1 = 1.0 - m0
        acc = jnp.zeros((qb, d), jnp.float32)
        for n in range(nbrk):
            vn = (jnp.dot(hn_ref[n].astype(jnp.bfloat16), vwn16,
                          preferred_element_type=jnp.float32)
                  + jnp.dot(ef_ref[n], vwe, preferred_element_type=jnp.float32)
                  + tv)
            w = a0[:, n:n + 1] * m0 + a1[:, n:n + 1] * m1
            acc = acc + w * vn
        out_ref[...] = (jnp.dot(acc, ow_ref[...],
                                preferred_element_type=jnp.float32)
                        + ob_ref[...]).astype(out_dtype)

    full = lambda i: (0, 0)
    return pl.pallas_call(
        body,
        grid=(n3 // qb,),
        in_specs=[
            pl.BlockSpec((qb, d), lambda i: (i, 0)),
            pl.BlockSpec((qb, 1), lambda i: (i, 0)),
            pl.BlockSpec((nbrk, qb, d), lambda i: (0, i, 0)),
            pl.BlockSpec((nbrk, qb, de), lambda i: (0, i, 0)),
            pl.BlockSpec((qb, nbrk), lambda i: (i, 0)),
            pl.BlockSpec((d, d), full),
            pl.BlockSpec((d, d), full),
            pl.BlockSpec((de, d), full),
            pl.BlockSpec((d, d), full),
            pl.BlockSpec((de, d), full),
            pl.BlockSpec((1, d), full),
            pl.BlockSpec((1, d), full),
            pl.BlockSpec((1, d), full),
            pl.BlockSpec((d, d), full),
            pl.BlockSpec((1, d), full),
            pl.BlockSpec((1, d), full),
            pl.BlockSpec((1, d), full),
        ],
        out_specs=pl.BlockSpec((qb, d), lambda i: (i, 0)),
        out_shape=jax.ShapeDtypeStruct((n3, d), out_dtype),
    )(h, ts2, hn_t, ef_t, nbr, qwT, kwnT, kweT, vwnT, vweT, bq, bk, bv, owT,
      ob, tw, tb)


def kernel(source_nodes, destination_nodes, negative_nodes, edge_times,
           edge_idxs, neighbors, nbr_edge_idxs, n_neighbors, node_features,
           edge_features, time_w, time_b, q_w, k_w, v_w, in_b, out_w, out_b):
    b = source_nodes.shape[0]
    n3 = 3 * b
    nbrk = neighbors.shape[1]
    v, d = node_features.shape
    de = edge_features.shape[1]

    nodes = jnp.concatenate([source_nodes, destination_nodes, negative_nodes])
    nodes = nodes.astype(jnp.int32)
    ts2 = jnp.concatenate([edge_times, edge_times, edge_times])[:, None]

    vp = ((v + NW * LANES - 1) // (NW * LANES)) * NW * LANES
    nf_pad = jnp.concatenate(
        [node_features, jnp.zeros((vp - v, d), jnp.float32)], axis=0)

    nbr_t_flat = neighbors.astype(jnp.int32).T.reshape(-1)
    idx_all = jnp.concatenate([nodes, nbr_t_flat])
    ef_idx = nbr_edge_idxs.astype(jnp.int32).T.reshape(-1)

    qwT = q_w.T
    kwnT = k_w[:, :d].T
    kweT = k_w[:, d:].T
    vwnT = v_w[:, :d].T
    vweT = v_w[:, d:].T
    bq = in_b[:d][None, :]
    bk = in_b[d:2 * d][None, :]
    bv = in_b[2 * d:][None, :]
    ob = out_b[None, :]
    tw = time_w[:, 0][None, :]
    tb = time_b[None, :]

    g1 = _gather_rows(nf_pad, idx_all, chunk=56, nbuf=6)
    efg = _gather_rows(edge_features, ef_idx, chunk=96, nbuf=4, tc_tiling=False)
    h0 = g1[:n3]
    hn_t = g1[n3:].reshape(nbrk, n3, d)
    ef_t = efg.reshape(nbrk, n3, de)

    h1 = _tc_layer(h0, ts2, hn_t, ef_t, neighbors, qwT, kwnT, kweT, vwnT,
                   vweT, bq, bk, bv, out_w.T, ob, tw, tb)

    nf2 = _scatter_update(nf_pad, nodes, h1)
    g2 = _gather_rows(nf2, idx_all, chunk=56, nbuf=6)
    h0b = g2[:n3]
    hn2_t = g2[n3:].reshape(nbrk, n3, d)

    h2 = _tc_layer(h0b, ts2, hn2_t, ef_t, neighbors, qwT, kwnT, kweT, vwnT,
                   vweT, bq, bk, bv, out_w.T, ob, tw, tb)

    return (h2[:b], h2[b:2 * b], h2[2 * b:])
